# Initial kernel scaffold; baseline (speedup 1.0000x reference)
import jax
import jax.numpy as jnp
from jax.experimental import pallas as pl


def kernel(probs, alias, index, prob):
    raise NotImplementedError("write your pallas kernel here")


# speedup vs baseline: 290.2382x; 290.2382x over previous
import functools

import jax
import jax.numpy as jnp
from jax import lax
from jax.experimental import pallas as pl
from jax.experimental.pallas import tpu as pltpu
from jax.experimental.pallas import tpu_sc as plsc

VOCAB_PAD = 1024

NC = 2
NS = 16
NW = NC * NS

N = 16384 * 200
PER_W = N // NW
CHUNK = 12800
NCHUNK = PER_W // CHUNK
L = 16


def _body(probs_hbm, alias_hbm, index_hbm, prob_hbm, out_hbm,
          probs_v, alias_v, idx_v, prob_v, out_v):
    wid = lax.axis_index("s") * NC + lax.axis_index("c")
    base = wid * PER_W

    pltpu.sync_copy(probs_hbm, probs_v)
    pltpu.sync_copy(alias_hbm, alias_v)

    for j in range(NCHUNK):
        off = base + j * CHUNK
        pltpu.sync_copy(index_hbm.at[pl.ds(off, CHUNK)], idx_v)
        pltpu.sync_copy(prob_hbm.at[pl.ds(off, CHUNK)], prob_v)

        def inner(i, _):
            s = pl.ds(i * L, L)
            idx = idx_v[s]
            pv = prob_v[s]
            pa = plsc.load_gather(probs_v, [idx])
            al = plsc.load_gather(alias_v, [idx])
            out_v[s] = jnp.where(pv < pa, idx, al)
            return 0

        lax.fori_loop(0, CHUNK // L, inner, 0)
        pltpu.sync_copy(out_v, out_hbm.at[pl.ds(off, CHUNK)])


@jax.jit
def _sample(probs_pad, alias_pad, index_flat, prob_flat):
    mesh = plsc.VectorSubcoreMesh(core_axis_name="c", subcore_axis_name="s")
    return pl.kernel(
        _body,
        out_type=jax.ShapeDtypeStruct((N,), jnp.int32),
        mesh=mesh,
        scratch_types=[
            pltpu.VMEM((VOCAB_PAD,), jnp.float32),
            pltpu.VMEM((VOCAB_PAD,), jnp.int32),
            pltpu.VMEM((CHUNK,), jnp.int32),
            pltpu.VMEM((CHUNK,), jnp.float32),
            pltpu.VMEM((CHUNK,), jnp.int32),
        ],
        compiler_params=pltpu.CompilerParams(needs_layout_passes=False),
    )(probs_pad, alias_pad, index_flat, prob_flat)


def kernel(probs, alias, index, prob):
    v = probs.shape[0]
    probs_pad = jnp.pad(probs, (0, VOCAB_PAD - v))
    alias_pad = jnp.pad(alias, (0, VOCAB_PAD - v))
    out = _sample(probs_pad, alias_pad, index.reshape(-1), prob.reshape(-1))
    return out.reshape(index.shape)


# speedup vs baseline: 322.2851x; 1.1104x over previous
import functools

import jax
import jax.numpy as jnp
from jax import lax
from jax.experimental import pallas as pl
from jax.experimental.pallas import tpu as pltpu
from jax.experimental.pallas import tpu_sc as plsc

VOCAB_PAD = 1024

NC = 2
NS = 16
NW = NC * NS

N = 16384 * 200
PER_W = N // NW
CHUNK = 12800
NCHUNK = PER_W // CHUNK
L = 16


UNROLL = 8


def _body(probs_hbm, alias_hbm, index_hbm, prob_hbm, out_hbm,
          probs_v, alias_v, idx_v0, idx_v1, prob_v0, prob_v1, out_v0, out_v1,
          si0, si1, sp0, sp1, so0, so1):
    wid = lax.axis_index("s") * NC + lax.axis_index("c")
    base = wid * PER_W

    pltpu.sync_copy(probs_hbm, probs_v)
    pltpu.sync_copy(alias_hbm, alias_v)

    si = (si0, si1)
    sp = (sp0, sp1)
    so = (so0, so1)
    idx_b = (idx_v0, idx_v1)
    prob_b = (prob_v0, prob_v1)
    out_b = (out_v0, out_v1)
    in_desc = [None, None]
    out_desc = [None, None]

    in_desc[0] = (
        pltpu.async_copy(index_hbm.at[pl.ds(base, CHUNK)], idx_b[0], si[0]),
        pltpu.async_copy(prob_hbm.at[pl.ds(base, CHUNK)], prob_b[0], sp[0]),
    )

    for j in range(NCHUNK):
        buf = j % 2
        nxt = 1 - buf
        if j + 1 < NCHUNK:
            offn = base + (j + 1) * CHUNK
            in_desc[nxt] = (
                pltpu.async_copy(index_hbm.at[pl.ds(offn, CHUNK)],
                                 idx_b[nxt], si[nxt]),
                pltpu.async_copy(prob_hbm.at[pl.ds(offn, CHUNK)],
                                 prob_b[nxt], sp[nxt]),
            )
        di, dp = in_desc[buf]
        di.wait()
        dp.wait()
        if out_desc[buf] is not None:
            out_desc[buf].wait()

        ib = idx_b[buf]
        pb = prob_b[buf]
        ob = out_b[buf]

        def inner(i, _):
            b0 = i * (L * UNROLL)
            for u in range(UNROLL):
                s = pl.ds(b0 + u * L, L)
                idx = ib[s]
                pv = pb[s]
                pa = plsc.load_gather(probs_v, [idx])
                al = plsc.load_gather(alias_v, [idx])
                ob[s] = jnp.where(pv < pa, idx, al)
            return 0

        lax.fori_loop(0, CHUNK // (L * UNROLL), inner, 0)
        out_desc[buf] = pltpu.async_copy(
            ob, out_hbm.at[pl.ds(base + j * CHUNK, CHUNK)], so[buf])

    out_desc[0].wait()
    out_desc[1].wait()


@jax.jit
def _sample(probs_pad, alias_pad, index_flat, prob_flat):
    mesh = plsc.VectorSubcoreMesh(core_axis_name="c", subcore_axis_name="s")
    return pl.kernel(
        _body,
        out_type=jax.ShapeDtypeStruct((N,), jnp.int32),
        mesh=mesh,
        scratch_types=[
            pltpu.VMEM((VOCAB_PAD,), jnp.float32),
            pltpu.VMEM((VOCAB_PAD,), jnp.int32),
            pltpu.VMEM((CHUNK,), jnp.int32),
            pltpu.VMEM((CHUNK,), jnp.int32),
            pltpu.VMEM((CHUNK,), jnp.float32),
            pltpu.VMEM((CHUNK,), jnp.float32),
            pltpu.VMEM((CHUNK,), jnp.int32),
            pltpu.VMEM((CHUNK,), jnp.int32),
            pltpu.SemaphoreType.DMA,
            pltpu.SemaphoreType.DMA,
            pltpu.SemaphoreType.DMA,
            pltpu.SemaphoreType.DMA,
            pltpu.SemaphoreType.DMA,
            pltpu.SemaphoreType.DMA,
        ],
        compiler_params=pltpu.CompilerParams(needs_layout_passes=False),
    )(probs_pad, alias_pad, index_flat, prob_flat)


def kernel(probs, alias, index, prob):
    v = probs.shape[0]
    probs_pad = jnp.pad(probs, (0, VOCAB_PAD - v))
    alias_pad = jnp.pad(alias, (0, VOCAB_PAD - v))
    out = _sample(probs_pad, alias_pad, index.reshape(-1), prob.reshape(-1))
    return out.reshape(index.shape)


# speedup vs baseline: 495.9532x; 1.5389x over previous
import jax
import jax.numpy as jnp
from jax import lax
from jax.experimental import pallas as pl
from jax.experimental.pallas import tpu as pltpu
from jax.experimental.pallas import tpu_sc as plsc

VOCAB_PAD = 1024

NC = 2
NS = 16
NW = NC * NS

R = 16384
C = 200
PER_W = R // NW
RBLK = 64
NCHUNK = PER_W // RBLK
L = 16

COLS = tuple(range(0, C - L + 1, L)) + ((C - L),)


def _body(probs_hbm, alias_hbm, index_hbm, prob_hbm, out_hbm,
          probs_v, alias_v, idx_v0, idx_v1, prob_v0, prob_v1, out_v0, out_v1,
          si0, si1, sp0, sp1, so0, so1):
    wid = lax.axis_index("s") * NC + lax.axis_index("c")
    base = wid * PER_W

    pltpu.sync_copy(probs_hbm, probs_v)
    pltpu.sync_copy(alias_hbm, alias_v)

    si = (si0, si1)
    sp = (sp0, sp1)
    so = (so0, so1)
    idx_b = (idx_v0, idx_v1)
    prob_b = (prob_v0, prob_v1)
    out_b = (out_v0, out_v1)
    in_desc = [None, None]
    out_desc = [None, None]

    in_desc[0] = (
        pltpu.async_copy(index_hbm.at[pl.ds(base, RBLK), :], idx_b[0], si[0]),
        pltpu.async_copy(prob_hbm.at[pl.ds(base, RBLK), :], prob_b[0], sp[0]),
    )

    for j in range(NCHUNK):
        buf = j % 2
        nxt = 1 - buf
        if j + 1 < NCHUNK:
            rn = base + (j + 1) * RBLK
            in_desc[nxt] = (
                pltpu.async_copy(index_hbm.at[pl.ds(rn, RBLK), :],
                                 idx_b[nxt], si[nxt]),
                pltpu.async_copy(prob_hbm.at[pl.ds(rn, RBLK), :],
                                 prob_b[nxt], sp[nxt]),
            )
        di, dp = in_desc[buf]
        di.wait()
        dp.wait()
        if out_desc[buf] is not None:
            out_desc[buf].wait()

        ib = idx_b[buf]
        pb = prob_b[buf]
        ob = out_b[buf]

        def inner(r, _):
            for c in COLS:
                s = pl.ds(c, L)
                idx = ib[r, s]
                pv = pb[r, s]
                pa = plsc.load_gather(probs_v, [idx])
                al = plsc.load_gather(alias_v, [idx])
                ob[r, s] = jnp.where(pv < pa, idx, al)
            return 0

        lax.fori_loop(0, RBLK, inner, 0)
        out_desc[buf] = pltpu.async_copy(
            ob, out_hbm.at[pl.ds(base + j * RBLK, RBLK), :], so[buf])

    out_desc[0].wait()
    out_desc[1].wait()


@jax.jit
def _sample(probs_pad, alias_pad, index, prob):
    mesh = plsc.VectorSubcoreMesh(core_axis_name="c", subcore_axis_name="s")
    return pl.kernel(
        _body,
        out_type=jax.ShapeDtypeStruct((R, C), jnp.int32),
        mesh=mesh,
        scratch_types=[
            pltpu.VMEM((VOCAB_PAD,), jnp.float32),
            pltpu.VMEM((VOCAB_PAD,), jnp.int32),
            pltpu.VMEM((RBLK, C), jnp.int32),
            pltpu.VMEM((RBLK, C), jnp.int32),
            pltpu.VMEM((RBLK, C), jnp.float32),
            pltpu.VMEM((RBLK, C), jnp.float32),
            pltpu.VMEM((RBLK, C), jnp.int32),
            pltpu.VMEM((RBLK, C), jnp.int32),
            pltpu.SemaphoreType.DMA,
            pltpu.SemaphoreType.DMA,
            pltpu.SemaphoreType.DMA,
            pltpu.SemaphoreType.DMA,
            pltpu.SemaphoreType.DMA,
            pltpu.SemaphoreType.DMA,
        ],
        compiler_params=pltpu.CompilerParams(needs_layout_passes=False),
    )(probs_pad, alias_pad, index, prob)


def kernel(probs, alias, index, prob):
    v = probs.shape[0]
    probs_pad = jnp.pad(probs, (0, VOCAB_PAD - v))
    alias_pad = jnp.pad(alias, (0, VOCAB_PAD - v))
    return _sample(probs_pad, alias_pad, index, prob)
